# baseline (speedup 1.0000x reference)
import functools

import jax
import jax.numpy as jnp
from jax import lax
from jax.experimental import pallas as pl
from jax.experimental.pallas import tpu as pltpu
from jax.experimental.pallas import tpu_sc as plsc

B, T, S = 16, 2048, 8
D = 64
A = 1000000
N = B * T
NC, NS, L = 2, 16, 16
NW = NC * NS
TPW = N // NW
C = 32
NCHUNK = TPW // C
FBLK = 2048


def _fmt_body(tbl_t_ref, out_ref):
    blk = tbl_t_ref[...]
    out_ref[:, 0:D] = blk.T
    out_ref[:, D:2 * D] = jnp.zeros((FBLK, D), jnp.float32)


@functools.partial(jax.jit, donate_argnums=())
def _sac(idx_flat, amap_t_flat, syn_table_t, pos_flat):
    table_pad = pl.pallas_call(
        _fmt_body,
        grid=((A + FBLK - 1) // FBLK,),
        in_specs=[pl.BlockSpec((D, FBLK), lambda i: (0, i))],
        out_specs=pl.BlockSpec((FBLK, 2 * D), lambda i: (i, 0)),
        out_shape=jax.ShapeDtypeStruct((A, 2 * D), jnp.float32),
        compiler_params=pltpu.CompilerParams(
            dimension_semantics=("parallel",)),
    )(syn_table_t)

    mesh = plsc.VectorSubcoreMesh(core_axis_name="c", subcore_axis_name="s")
    return pl.kernel(
        _sac_body,
        out_type=(jax.ShapeDtypeStruct((N * D,), jnp.float32),
                  jax.ShapeDtypeStruct((N * S,), jnp.int32)),
        mesh=mesh,
        compiler_params=pltpu.CompilerParams(use_tc_tiling_on_sc=True),
        scratch_types=[
            pltpu.VMEM((TPW,), jnp.int32),
            pltpu.VMEM((TPW * S,), jnp.int32),
            pltpu.VMEM((TPW * S,), jnp.int32),
            pltpu.VMEM((C * S, 2 * D), jnp.float32),
            pltpu.VMEM((C * S, 2 * D), jnp.float32),
            pltpu.VMEM((C * D,), jnp.float32),
            pltpu.VMEM((C * D,), jnp.float32),
            pltpu.SemaphoreType.DMA,
            pltpu.SemaphoreType.DMA,
            pltpu.SemaphoreType.DMA,
            pltpu.SemaphoreType.DMA,
            pltpu.SemaphoreType.DMA,
        ],
    )(idx_flat, amap_t_flat, table_pad, pos_flat)


def _sac_body(idx_hbm, amap_t_hbm, table_hbm, pos_flat_hbm,
              out_hbm, syn_hbm,
              idx_v, offs_v, syn_v, rows0_v, rows1_v, pos_v, out_v,
              sem_syn_in, sem_syn_out, sem_rows0, sem_rows1, sem_out):
    wid = lax.axis_index("s") * NC + lax.axis_index("c")
    base = wid * TPW
    pos_base = (wid % 2) * TPW

    pltpu.sync_copy(idx_hbm.at[pl.ds(base, TPW)], idx_v)

    lane = lax.iota(jnp.int32, L)
    pat0 = lax.shift_right_logical(lane, 3)
    joff = lax.bitwise_and(lane, S - 1) * 100000

    def do_expand(g, _):
        iv = idx_v[pl.ds(g * L, L)]
        for h in range(L // 2):
            tok = jnp.take_along_axis(iv, pat0 + 2 * h, axis=0,
                                      mode="promise_in_bounds")
            offs_v[pl.ds((g * L + 2 * h) * S, L)] = tok + joff
        return 0

    lax.fori_loop(0, TPW // L, do_expand, 0)

    pltpu.async_copy(amap_t_hbm.at[offs_v], syn_v, sem_syn_in).wait()
    syn_out = pltpu.make_async_copy(
        syn_v, syn_hbm.at[pl.ds(base * S, TPW * S)], sem_syn_out)
    syn_out.start()

    def start_rows(c, buf, sem):
        pltpu.async_copy(table_hbm.at[syn_v.at[pl.ds(c * C * S, C * S)]],
                         buf, sem)

    start_rows(0, rows0_v, sem_rows0)

    def do_chunk(c, rows_v, sem, nxt_buf, nxt_sem):
        pltpu.make_async_copy(
            table_hbm.at[syn_v.at[pl.ds(c * C * S, C * S)]], rows_v,
            sem).wait()

        @pl.when(c + 1 < NCHUNK)
        def _():
            start_rows(c + 1, nxt_buf, nxt_sem)

        pltpu.sync_copy(pos_flat_hbm.at[pl.ds((pos_base + c * C) * D, C * D)],
                        pos_v)

        def do_token(t, _):
            t8 = t * S
            for k in range(D // L):
                sl = pl.ds(k * L, L)
                a0 = rows_v[t8 + 0, sl] + rows_v[t8 + 1, sl]
                a1 = rows_v[t8 + 2, sl] + rows_v[t8 + 3, sl]
                a2 = rows_v[t8 + 4, sl] + rows_v[t8 + 5, sl]
                a3 = rows_v[t8 + 6, sl] + rows_v[t8 + 7, sl]
                acc = (a0 + a1) + (a2 + a3)
                out_v[pl.ds(t * D + k * L, L)] = (
                    acc * 0.125 + pos_v[pl.ds(t * D + k * L, L)])
            return 0

        lax.fori_loop(0, C, do_token, 0)
        pltpu.sync_copy(out_v, out_hbm.at[pl.ds((base + c * C) * D, C * D)])

    def loop_body(g, _):
        c = g * 2
        do_chunk(c, rows0_v, sem_rows0, rows1_v, sem_rows1)
        do_chunk(c + 1, rows1_v, sem_rows1, rows0_v, sem_rows0)
        return 0

    lax.fori_loop(0, NCHUNK // 2, loop_body, 0)
    syn_out.wait()


def kernel(idx, address_map, syn_table, pos_table):
    out_flat, syn_flat = _sac(idx.reshape(-1), address_map.T.reshape(-1),
                              syn_table.T, pos_table.reshape(-1))
    return out_flat.reshape(B, T, D), syn_flat.reshape(B, T, S)


# speedup vs baseline: 1.2464x; 1.2464x over previous
import functools

import jax
import jax.numpy as jnp
from jax import lax
from jax.experimental import pallas as pl
from jax.experimental.pallas import tpu as pltpu
from jax.experimental.pallas import tpu_sc as plsc

B, T, S = 16, 2048, 8
D = 64
A = 1000000
N = B * T
NC, NS, L = 2, 16, 16
NW = NC * NS
TPW = N // NW
C = 32
NCHUNK = TPW // C
FBLK = 4096


def _fmt_body(tbl_t_ref, out_ref):
    blk = tbl_t_ref[...]
    out_ref[:, 0:D] = blk.T
    out_ref[:, D:2 * D] = jnp.zeros((FBLK, D), jnp.float32)


@functools.partial(jax.jit, donate_argnums=())
def _sac(idx_flat, amap_t_flat, syn_table_t, pos_flat):
    table_pad = pl.pallas_call(
        _fmt_body,
        grid=((A + FBLK - 1) // FBLK,),
        in_specs=[pl.BlockSpec((D, FBLK), lambda i: (0, i))],
        out_specs=pl.BlockSpec((FBLK, 2 * D), lambda i: (i, 0)),
        out_shape=jax.ShapeDtypeStruct((A, 2 * D), jnp.float32),
        compiler_params=pltpu.CompilerParams(
            dimension_semantics=("parallel",)),
    )(syn_table_t)

    mesh = plsc.VectorSubcoreMesh(core_axis_name="c", subcore_axis_name="s")
    return pl.kernel(
        _sac_body,
        out_type=(jax.ShapeDtypeStruct((N * D,), jnp.float32),
                  jax.ShapeDtypeStruct((N * S,), jnp.int32)),
        mesh=mesh,
        compiler_params=pltpu.CompilerParams(use_tc_tiling_on_sc=True),
        scratch_types=[
            pltpu.VMEM((TPW,), jnp.int32),
            pltpu.VMEM((TPW * S,), jnp.int32),
            pltpu.VMEM((TPW * S,), jnp.int32),
            pltpu.VMEM((C * S, 2 * D), jnp.float32),
            pltpu.VMEM((C * S, 2 * D), jnp.float32),
            pltpu.VMEM((C * D,), jnp.float32),
            pltpu.VMEM((C * D,), jnp.float32),
            pltpu.SemaphoreType.DMA,
            pltpu.SemaphoreType.DMA,
            pltpu.SemaphoreType.DMA,
            pltpu.SemaphoreType.DMA,
            pltpu.SemaphoreType.DMA,
        ],
    )(idx_flat, amap_t_flat, table_pad, pos_flat)


def _sac_body(idx_hbm, amap_t_hbm, table_hbm, pos_flat_hbm,
              out_hbm, syn_hbm,
              idx_v, offs_v, syn_v, rows0_v, rows1_v, pos_v, out_v,
              sem_syn_in, sem_syn_out, sem_rows0, sem_rows1, sem_out):
    wid = lax.axis_index("s") * NC + lax.axis_index("c")
    base = wid * TPW
    pos_base = (wid % 2) * TPW

    pltpu.sync_copy(idx_hbm.at[pl.ds(base, TPW)], idx_v)

    lane = lax.iota(jnp.int32, L)
    pat0 = lax.shift_right_logical(lane, 3)
    joff = lax.bitwise_and(lane, S - 1) * 100000

    def do_expand(g, _):
        iv = idx_v[pl.ds(g * L, L)]
        for h in range(L // 2):
            tok = jnp.take_along_axis(iv, pat0 + 2 * h, axis=0,
                                      mode="promise_in_bounds")
            offs_v[pl.ds((g * L + 2 * h) * S, L)] = tok + joff
        return 0

    lax.fori_loop(0, TPW // L, do_expand, 0)

    pltpu.async_copy(amap_t_hbm.at[offs_v], syn_v, sem_syn_in).wait()
    syn_out = pltpu.make_async_copy(
        syn_v, syn_hbm.at[pl.ds(base * S, TPW * S)], sem_syn_out)
    syn_out.start()

    def start_rows(c, buf, sem):
        pltpu.async_copy(table_hbm.at[syn_v.at[pl.ds(c * C * S, C * S)]],
                         buf, sem)

    start_rows(0, rows0_v, sem_rows0)

    def do_chunk(c, rows_v, sem, nxt_buf, nxt_sem):
        pltpu.make_async_copy(
            table_hbm.at[syn_v.at[pl.ds(c * C * S, C * S)]], rows_v,
            sem).wait()

        @pl.when(c + 1 < NCHUNK)
        def _():
            start_rows(c + 1, nxt_buf, nxt_sem)

        pltpu.sync_copy(pos_flat_hbm.at[pl.ds((pos_base + c * C) * D, C * D)],
                        pos_v)

        def do_token(t, _):
            t8 = t * S
            for k in range(D // L):
                sl = pl.ds(k * L, L)
                a0 = rows_v[t8 + 0, sl] + rows_v[t8 + 1, sl]
                a1 = rows_v[t8 + 2, sl] + rows_v[t8 + 3, sl]
                a2 = rows_v[t8 + 4, sl] + rows_v[t8 + 5, sl]
                a3 = rows_v[t8 + 6, sl] + rows_v[t8 + 7, sl]
                acc = (a0 + a1) + (a2 + a3)
                out_v[pl.ds(t * D + k * L, L)] = (
                    acc * 0.125 + pos_v[pl.ds(t * D + k * L, L)])
            return 0

        lax.fori_loop(0, C, do_token, 0)
        pltpu.sync_copy(out_v, out_hbm.at[pl.ds((base + c * C) * D, C * D)])

    def loop_body(g, _):
        c = g * 2
        do_chunk(c, rows0_v, sem_rows0, rows1_v, sem_rows1)
        do_chunk(c + 1, rows1_v, sem_rows1, rows0_v, sem_rows0)
        return 0

    lax.fori_loop(0, NCHUNK // 2, loop_body, 0)
    syn_out.wait()


def kernel(idx, address_map, syn_table, pos_table):
    out_flat, syn_flat = _sac(idx.reshape(-1), address_map.T.reshape(-1),
                              syn_table.T, pos_table.reshape(-1))
    return out_flat.reshape(B, T, D), syn_flat.reshape(B, T, S)


# speedup vs baseline: 1.5305x; 1.2279x over previous
import functools

import jax
import jax.numpy as jnp
from jax import lax
from jax.experimental import pallas as pl
from jax.experimental.pallas import tpu as pltpu
from jax.experimental.pallas import tpu_sc as plsc

B, T, S = 16, 2048, 8
D = 64
A = 1000000
N = B * T
NC, NS, L = 2, 16, 16
NW = NC * NS
TPW = N // NW
C = 32
NCHUNK = TPW // C
FBLK = 16384


def _fmt_body(tbl_t_ref, out_ref):
    blk = tbl_t_ref[...]
    out_ref[:, 0:D] = blk.T
    out_ref[:, D:2 * D] = jnp.zeros((FBLK, D), jnp.float32)


@functools.partial(jax.jit, donate_argnums=())
def _sac(idx_flat, amap_t_flat, syn_table_t, pos_flat):
    table_pad = pl.pallas_call(
        _fmt_body,
        grid=((A + FBLK - 1) // FBLK,),
        in_specs=[pl.BlockSpec((D, FBLK), lambda i: (0, i))],
        out_specs=pl.BlockSpec((FBLK, 2 * D), lambda i: (i, 0)),
        out_shape=jax.ShapeDtypeStruct((A, 2 * D), jnp.float32),
        compiler_params=pltpu.CompilerParams(
            dimension_semantics=("parallel",)),
    )(syn_table_t)

    mesh = plsc.VectorSubcoreMesh(core_axis_name="c", subcore_axis_name="s")
    return pl.kernel(
        _sac_body,
        out_type=(jax.ShapeDtypeStruct((N * D,), jnp.float32),
                  jax.ShapeDtypeStruct((N * S,), jnp.int32)),
        mesh=mesh,
        compiler_params=pltpu.CompilerParams(use_tc_tiling_on_sc=True),
        scratch_types=[
            pltpu.VMEM((TPW,), jnp.int32),
            pltpu.VMEM((TPW * S,), jnp.int32),
            pltpu.VMEM((TPW * S,), jnp.int32),
            pltpu.VMEM((C * S, 2 * D), jnp.float32),
            pltpu.VMEM((C * S, 2 * D), jnp.float32),
            pltpu.VMEM((C * D,), jnp.float32),
            pltpu.VMEM((C * D,), jnp.float32),
            pltpu.SemaphoreType.DMA,
            pltpu.SemaphoreType.DMA,
            pltpu.SemaphoreType.DMA,
            pltpu.SemaphoreType.DMA,
            pltpu.SemaphoreType.DMA,
        ],
    )(idx_flat, amap_t_flat, table_pad, pos_flat)


def _sac_body(idx_hbm, amap_t_hbm, table_hbm, pos_flat_hbm,
              out_hbm, syn_hbm,
              idx_v, offs_v, syn_v, rows0_v, rows1_v, pos_v, out_v,
              sem_syn_in, sem_syn_out, sem_rows0, sem_rows1, sem_out):
    wid = lax.axis_index("s") * NC + lax.axis_index("c")
    base = wid * TPW
    pos_base = (wid % 2) * TPW

    pltpu.sync_copy(idx_hbm.at[pl.ds(base, TPW)], idx_v)

    lane = lax.iota(jnp.int32, L)
    pat0 = lax.shift_right_logical(lane, 3)
    joff = lax.bitwise_and(lane, S - 1) * 100000

    def do_expand(g, _):
        iv = idx_v[pl.ds(g * L, L)]
        for h in range(L // 2):
            tok = jnp.take_along_axis(iv, pat0 + 2 * h, axis=0,
                                      mode="promise_in_bounds")
            offs_v[pl.ds((g * L + 2 * h) * S, L)] = tok + joff
        return 0

    lax.fori_loop(0, TPW // L, do_expand, 0)

    pltpu.async_copy(amap_t_hbm.at[offs_v], syn_v, sem_syn_in).wait()
    syn_out = pltpu.make_async_copy(
        syn_v, syn_hbm.at[pl.ds(base * S, TPW * S)], sem_syn_out)
    syn_out.start()

    def start_rows(c, buf, sem):
        pltpu.async_copy(table_hbm.at[syn_v.at[pl.ds(c * C * S, C * S)]],
                         buf, sem)

    start_rows(0, rows0_v, sem_rows0)

    def do_chunk(c, rows_v, sem, nxt_buf, nxt_sem):
        pltpu.make_async_copy(
            table_hbm.at[syn_v.at[pl.ds(c * C * S, C * S)]], rows_v,
            sem).wait()

        @pl.when(c + 1 < NCHUNK)
        def _():
            start_rows(c + 1, nxt_buf, nxt_sem)

        pltpu.sync_copy(pos_flat_hbm.at[pl.ds((pos_base + c * C) * D, C * D)],
                        pos_v)

        def do_token(t, _):
            t8 = t * S
            for k in range(D // L):
                sl = pl.ds(k * L, L)
                a0 = rows_v[t8 + 0, sl] + rows_v[t8 + 1, sl]
                a1 = rows_v[t8 + 2, sl] + rows_v[t8 + 3, sl]
                a2 = rows_v[t8 + 4, sl] + rows_v[t8 + 5, sl]
                a3 = rows_v[t8 + 6, sl] + rows_v[t8 + 7, sl]
                acc = (a0 + a1) + (a2 + a3)
                out_v[pl.ds(t * D + k * L, L)] = (
                    acc * 0.125 + pos_v[pl.ds(t * D + k * L, L)])
            return 0

        lax.fori_loop(0, C, do_token, 0)
        pltpu.sync_copy(out_v, out_hbm.at[pl.ds((base + c * C) * D, C * D)])

    def loop_body(g, _):
        c = g * 2
        do_chunk(c, rows0_v, sem_rows0, rows1_v, sem_rows1)
        do_chunk(c + 1, rows1_v, sem_rows1, rows0_v, sem_rows0)
        return 0

    lax.fori_loop(0, NCHUNK // 2, loop_body, 0)
    syn_out.wait()


def kernel(idx, address_map, syn_table, pos_table):
    out_flat, syn_flat = _sac(idx.reshape(-1), address_map.T.reshape(-1),
                              syn_table.T, pos_table.reshape(-1))
    return out_flat.reshape(B, T, D), syn_flat.reshape(B, T, S)


# speedup vs baseline: 1.5542x; 1.0155x over previous
import functools

import jax
import jax.numpy as jnp
from jax import lax
from jax.experimental import pallas as pl
from jax.experimental.pallas import tpu as pltpu
from jax.experimental.pallas import tpu_sc as plsc

B, T, S = 16, 2048, 8
D = 64
A = 1000000
N = B * T
NC, NS, L = 2, 16, 16
NW = NC * NS
TPW = N // NW
C = 32
NCHUNK = TPW // C
FBLK = 32768


def _fmt_body(tbl_t_ref, out_ref):
    blk = tbl_t_ref[...]
    out_ref[:, 0:D] = blk.T
    out_ref[:, D:2 * D] = jnp.zeros((FBLK, D), jnp.float32)


@functools.partial(jax.jit, donate_argnums=())
def _sac(idx_flat, amap_t_flat, syn_table_t, pos_flat):
    table_pad = pl.pallas_call(
        _fmt_body,
        grid=((A + FBLK - 1) // FBLK,),
        in_specs=[pl.BlockSpec((D, FBLK), lambda i: (0, i))],
        out_specs=pl.BlockSpec((FBLK, 2 * D), lambda i: (i, 0)),
        out_shape=jax.ShapeDtypeStruct((A, 2 * D), jnp.float32),
        compiler_params=pltpu.CompilerParams(
            dimension_semantics=("parallel",)),
    )(syn_table_t)

    mesh = plsc.VectorSubcoreMesh(core_axis_name="c", subcore_axis_name="s")
    return pl.kernel(
        _sac_body,
        out_type=(jax.ShapeDtypeStruct((N * D,), jnp.float32),
                  jax.ShapeDtypeStruct((N * S,), jnp.int32)),
        mesh=mesh,
        compiler_params=pltpu.CompilerParams(use_tc_tiling_on_sc=True),
        scratch_types=[
            pltpu.VMEM((TPW,), jnp.int32),
            pltpu.VMEM((TPW * S,), jnp.int32),
            pltpu.VMEM((TPW * S,), jnp.int32),
            pltpu.VMEM((C * S, 2 * D), jnp.float32),
            pltpu.VMEM((C * S, 2 * D), jnp.float32),
            pltpu.VMEM((C * D,), jnp.float32),
            pltpu.VMEM((C * D,), jnp.float32),
            pltpu.SemaphoreType.DMA,
            pltpu.SemaphoreType.DMA,
            pltpu.SemaphoreType.DMA,
            pltpu.SemaphoreType.DMA,
            pltpu.SemaphoreType.DMA,
        ],
    )(idx_flat, amap_t_flat, table_pad, pos_flat)


def _sac_body(idx_hbm, amap_t_hbm, table_hbm, pos_flat_hbm,
              out_hbm, syn_hbm,
              idx_v, offs_v, syn_v, rows0_v, rows1_v, pos_v, out_v,
              sem_syn_in, sem_syn_out, sem_rows0, sem_rows1, sem_out):
    wid = lax.axis_index("s") * NC + lax.axis_index("c")
    base = wid * TPW
    pos_base = (wid % 2) * TPW

    pltpu.sync_copy(idx_hbm.at[pl.ds(base, TPW)], idx_v)

    lane = lax.iota(jnp.int32, L)
    pat0 = lax.shift_right_logical(lane, 3)
    joff = lax.bitwise_and(lane, S - 1) * 100000

    def do_expand(g, _):
        iv = idx_v[pl.ds(g * L, L)]
        for h in range(L // 2):
            tok = jnp.take_along_axis(iv, pat0 + 2 * h, axis=0,
                                      mode="promise_in_bounds")
            offs_v[pl.ds((g * L + 2 * h) * S, L)] = tok + joff
        return 0

    lax.fori_loop(0, TPW // L, do_expand, 0)

    pltpu.async_copy(amap_t_hbm.at[offs_v], syn_v, sem_syn_in).wait()
    syn_out = pltpu.make_async_copy(
        syn_v, syn_hbm.at[pl.ds(base * S, TPW * S)], sem_syn_out)
    syn_out.start()

    def start_rows(c, buf, sem):
        pltpu.async_copy(table_hbm.at[syn_v.at[pl.ds(c * C * S, C * S)]],
                         buf, sem)

    start_rows(0, rows0_v, sem_rows0)

    def do_chunk(c, rows_v, sem, nxt_buf, nxt_sem):
        pltpu.make_async_copy(
            table_hbm.at[syn_v.at[pl.ds(c * C * S, C * S)]], rows_v,
            sem).wait()

        @pl.when(c + 1 < NCHUNK)
        def _():
            start_rows(c + 1, nxt_buf, nxt_sem)

        pltpu.sync_copy(pos_flat_hbm.at[pl.ds((pos_base + c * C) * D, C * D)],
                        pos_v)

        def do_token(t, _):
            t8 = t * S
            for k in range(D // L):
                sl = pl.ds(k * L, L)
                a0 = rows_v[t8 + 0, sl] + rows_v[t8 + 1, sl]
                a1 = rows_v[t8 + 2, sl] + rows_v[t8 + 3, sl]
                a2 = rows_v[t8 + 4, sl] + rows_v[t8 + 5, sl]
                a3 = rows_v[t8 + 6, sl] + rows_v[t8 + 7, sl]
                acc = (a0 + a1) + (a2 + a3)
                out_v[pl.ds(t * D + k * L, L)] = (
                    acc * 0.125 + pos_v[pl.ds(t * D + k * L, L)])
            return 0

        lax.fori_loop(0, C, do_token, 0)
        pltpu.sync_copy(out_v, out_hbm.at[pl.ds((base + c * C) * D, C * D)])

    def loop_body(g, _):
        c = g * 2
        do_chunk(c, rows0_v, sem_rows0, rows1_v, sem_rows1)
        do_chunk(c + 1, rows1_v, sem_rows1, rows0_v, sem_rows0)
        return 0

    lax.fori_loop(0, NCHUNK // 2, loop_body, 0)
    syn_out.wait()


def kernel(idx, address_map, syn_table, pos_table):
    out_flat, syn_flat = _sac(idx.reshape(-1), address_map.T.reshape(-1),
                              syn_table.T, pos_table.reshape(-1))
    return out_flat.reshape(B, T, D), syn_flat.reshape(B, T, S)
